# baseline (speedup 1.0000x reference)
import jax
import jax.numpy as jnp
from jax.experimental import pallas as pl
from jax.experimental.pallas import tpu as pltpu

_N, _P, _S = 16, 4096, 64
_ROWS = _N * _P // 2
_BLK = 2048


def _tent_body(d_ref, s_ref, o_ref):
    d = d_ref[...]
    sam = s_ref[...]
    x0 = d[:, 0:1]
    y0 = d[:, 1:2]
    x1 = d[:, 2:3]
    y1 = d[:, 3:4]
    m0 = jnp.broadcast_to(0.5 * (x0 + y0), (_BLK, _S))
    m1 = jnp.broadcast_to(0.5 * (x1 + y1), (_BLK, _S))
    h0 = jnp.broadcast_to(0.5 * (y0 - x0), (_BLK, _S))
    h1 = jnp.broadcast_to(0.5 * (y1 - x1), (_BLK, _S))
    m = jnp.concatenate([m0, m1], axis=1)
    h = jnp.concatenate([h0, h1], axis=1)
    o_ref[...] = jnp.maximum(h - jnp.abs(sam - m), 0.0)


def kernel(diagrams, samples):
    d4 = diagrams.reshape(_ROWS, 4)
    sam2 = jnp.concatenate([samples, samples])[None, :]
    out = pl.pallas_call(
        _tent_body,
        grid=(_ROWS // _BLK,),
        in_specs=[
            pl.BlockSpec((_BLK, 4), lambda i: (i, 0)),
            pl.BlockSpec((1, 2 * _S), lambda i: (0, 0)),
        ],
        out_specs=pl.BlockSpec((_BLK, 2 * _S), lambda i: (i, 0)),
        out_shape=jax.ShapeDtypeStruct((_ROWS, 2 * _S), jnp.float32),
    )(d4, sam2)
    return out.reshape(_N, _P, _S)
